# Initial kernel scaffold; baseline (speedup 1.0000x reference)
import jax
import jax.numpy as jnp
from jax.experimental import pallas as pl


def kernel(x, cb0, cb1, cb2, cb3):
    raise NotImplementedError("write your pallas kernel here")


# speedup vs baseline: 1.1488x; 1.1488x over previous
import jax
import jax.numpy as jnp
from jax.experimental import pallas as pl
from jax.experimental.pallas import tpu as pltpu

BLK = 16
N_VEC = 2048
DIMV = 256


def _vq_tc_kernel(flat_ref, cb_ref, q_ref, idx_ref, loss_ref, perp_ref):
    flat = flat_ref[...]
    cb = cb_ref[...]
    rn = jnp.sum(flat * flat, axis=1, keepdims=True)
    cn = jnp.sum(cb * cb, axis=1)[None, :]
    g = jnp.dot(flat, cb.T, preferred_element_type=jnp.float32)
    d = rn - 2.0 * g + cn
    idx = jnp.argmin(d, axis=1)
    onehot = (idx[:, None] == jax.lax.broadcasted_iota(jnp.int32, d.shape, 1)
              ).astype(jnp.float32)
    q = jnp.dot(onehot, cb, preferred_element_type=jnp.float32,
                precision=jax.lax.Precision.HIGHEST)
    q_ref[...] = q
    idx_ref[...] = idx[:, None]
    diff = q - flat
    loss_ref[...] = (1.25 * jnp.mean(diff * diff)).reshape(1, 1)
    counts = jnp.sum(onehot, axis=0, keepdims=True)
    avg = counts * (1.0 / N_VEC)
    perp_ref[...] = jnp.exp(-jnp.sum(avg * jnp.log(avg + 1e-10))).reshape(1, 1)


def _vq_channel(vec, cb, interpret=False):
    return pl.pallas_call(
        _vq_tc_kernel,
        out_shape=[
            jax.ShapeDtypeStruct((N_VEC, DIMV), jnp.float32),
            jax.ShapeDtypeStruct((N_VEC, 1), jnp.int32),
            jax.ShapeDtypeStruct((1, 1), jnp.float32),
            jax.ShapeDtypeStruct((1, 1), jnp.float32),
        ],
        interpret=interpret,
    )(vec, cb)


def kernel(x, cb0, cb1, cb2, cb3, interpret=False):
    cbs = [cb0, cb1, cb2, cb3]
    b, c, h, w = x.shape
    ht = h // BLK
    xh, idxs, losses, perps = [], [], [], []
    for i in range(c):
        vec = (x[:, i].reshape(b, ht, BLK, ht, BLK)
               .transpose(0, 1, 3, 2, 4).reshape(b * ht * ht, BLK * BLK))
        q, idx, loss, perp = _vq_channel(vec, cbs[i], interpret=interpret)
        xh.append(q.reshape(b, ht, ht, BLK, BLK)
                  .transpose(0, 1, 3, 2, 4).reshape(b, h, w))
        idxs.append(idx.reshape(b, 1, ht * ht))
        losses.append(loss.reshape(1))
        perps.append(perp.reshape(1))
    return (jnp.stack(xh, axis=1), jnp.concatenate(idxs, axis=1),
            jnp.concatenate(losses, axis=0), jnp.concatenate(perps, axis=0))


# speedup vs baseline: 1.2062x; 1.0500x over previous
import jax
import jax.numpy as jnp
from jax.experimental import pallas as pl
from jax.experimental.pallas import tpu as pltpu

BLK = 16
N_VEC = 2048
DIMV = 256


def _vq_tc_kernel(flat_ref, cb_ref, q_ref, idx_ref, loss_ref, perp_ref):
    flat = flat_ref[...]
    cb = cb_ref[...]
    rn = jnp.sum(flat * flat, axis=1, keepdims=True)
    cn = jnp.sum(cb * cb, axis=1)[None, :]
    g = jnp.dot(flat, cb.T, preferred_element_type=jnp.float32)
    d = rn - 2.0 * g + cn
    idx = jnp.argmin(d, axis=1)
    onehot = (idx[:, None] == jax.lax.broadcasted_iota(jnp.int32, d.shape, 1)
              ).astype(jnp.float32)
    q = jnp.dot(onehot, cb, preferred_element_type=jnp.float32)
    q_ref[...] = q
    idx_ref[...] = idx[:, None]
    diff = q - flat
    loss_ref[...] = (1.25 * jnp.mean(diff * diff)).reshape(1, 1)
    counts = jnp.sum(onehot, axis=0, keepdims=True)
    avg = counts * (1.0 / N_VEC)
    perp_ref[...] = jnp.exp(-jnp.sum(avg * jnp.log(avg + 1e-10))).reshape(1, 1)


def _vq_channel(vec, cb, interpret=False):
    return pl.pallas_call(
        _vq_tc_kernel,
        out_shape=[
            jax.ShapeDtypeStruct((N_VEC, DIMV), jnp.float32),
            jax.ShapeDtypeStruct((N_VEC, 1), jnp.int32),
            jax.ShapeDtypeStruct((1, 1), jnp.float32),
            jax.ShapeDtypeStruct((1, 1), jnp.float32),
        ],
        interpret=interpret,
    )(vec, cb)


def kernel(x, cb0, cb1, cb2, cb3, interpret=False):
    cbs = [cb0, cb1, cb2, cb3]
    b, c, h, w = x.shape
    ht = h // BLK
    xh, idxs, losses, perps = [], [], [], []
    for i in range(c):
        vec = (x[:, i].reshape(b, ht, BLK, ht, BLK)
               .transpose(0, 1, 3, 2, 4).reshape(b * ht * ht, BLK * BLK))
        q, idx, loss, perp = _vq_channel(vec, cbs[i], interpret=interpret)
        xh.append(q.reshape(b, ht, ht, BLK, BLK)
                  .transpose(0, 1, 3, 2, 4).reshape(b, h, w))
        idxs.append(idx.reshape(b, 1, ht * ht))
        losses.append(loss.reshape(1))
        perps.append(perp.reshape(1))
    return (jnp.stack(xh, axis=1), jnp.concatenate(idxs, axis=1),
            jnp.concatenate(losses, axis=0), jnp.concatenate(perps, axis=0))


# speedup vs baseline: 1.6678x; 1.3827x over previous
import jax
import jax.numpy as jnp
from jax.experimental import pallas as pl
from jax.experimental.pallas import tpu as pltpu

BLK = 16
N_VEC = 2048
DIMV = 256
KPAD = 1024
PAD_VAL = 1e4


def _vq_tc_kernel(flat_ref, cb_ref, q_ref, idx_ref, loss_ref, perp_ref):
    flat = flat_ref[0]
    cb = cb_ref[0]
    rn = jnp.sum(flat * flat, axis=1, keepdims=True)
    cn = jnp.sum(cb * cb, axis=1)[None, :]
    g = jnp.dot(flat, cb.T, preferred_element_type=jnp.float32)
    d = rn - 2.0 * g + cn
    idx = jnp.argmin(d, axis=1)
    onehot = (idx[:, None] == jax.lax.broadcasted_iota(jnp.int32, d.shape, 1)
              ).astype(jnp.float32)
    q = jnp.dot(onehot, cb, preferred_element_type=jnp.float32)
    q_ref[0] = q
    idx_ref[0] = idx[:, None]
    diff = q - flat
    loss_ref[...] = (1.25 * jnp.mean(diff * diff)).reshape(1, 1, 1)
    counts = jnp.sum(onehot, axis=0, keepdims=True)
    avg = counts * (1.0 / N_VEC)
    perp_ref[...] = jnp.exp(-jnp.sum(avg * jnp.log(avg + 1e-10))).reshape(1, 1, 1)


def kernel(x, cb0, cb1, cb2, cb3, interpret=False):
    b, c, h, w = x.shape
    ht = h // BLK
    cbs = []
    for cb in (cb0, cb1, cb2, cb3):
        k = cb.shape[0]
        if k < KPAD:
            cb = jnp.concatenate(
                [cb, jnp.full((KPAD - k, DIMV), PAD_VAL, cb.dtype)], axis=0)
        cbs.append(cb)
    cb_pad = jnp.stack(cbs, axis=0)

    vec = (x.reshape(b, c, ht, BLK, ht, BLK)
           .transpose(1, 0, 2, 4, 3, 5)
           .reshape(c, b * ht * ht, BLK * BLK))

    q, idx, loss, perp = pl.pallas_call(
        _vq_tc_kernel,
        grid=(c,),
        in_specs=[
            pl.BlockSpec((1, N_VEC, DIMV), lambda i: (i, 0, 0)),
            pl.BlockSpec((1, KPAD, DIMV), lambda i: (i, 0, 0)),
        ],
        out_specs=[
            pl.BlockSpec((1, N_VEC, DIMV), lambda i: (i, 0, 0)),
            pl.BlockSpec((1, N_VEC, 1), lambda i: (i, 0, 0)),
            pl.BlockSpec((1, 1, 1), lambda i: (i, 0, 0)),
            pl.BlockSpec((1, 1, 1), lambda i: (i, 0, 0)),
        ],
        out_shape=[
            jax.ShapeDtypeStruct((c, N_VEC, DIMV), jnp.float32),
            jax.ShapeDtypeStruct((c, N_VEC, 1), jnp.int32),
            jax.ShapeDtypeStruct((c, 1, 1), jnp.float32),
            jax.ShapeDtypeStruct((c, 1, 1), jnp.float32),
        ],
        interpret=interpret,
    )(vec, cb_pad)

    x_hat = (q.reshape(c, b, ht, ht, BLK, BLK)
             .transpose(1, 0, 2, 4, 3, 5)
             .reshape(b, c, h, w))
    indices = idx.reshape(c, b, ht * ht).transpose(1, 0, 2)
    return (x_hat, indices, loss.reshape(c), perp.reshape(c))


# speedup vs baseline: 2.3925x; 1.4345x over previous
import functools

import jax
import jax.numpy as jnp
from jax.experimental import pallas as pl
from jax.experimental.pallas import tpu as pltpu

BLK = 16
HT = 16
NV = 256
DIMV = 256
KPAD = 1024
NB = 8
NC = 4
PAD_VAL = 1e4


def _blockify(img):
    return (img.reshape(HT, BLK, HT, BLK).transpose(0, 2, 1, 3)
            .reshape(NV, DIMV))


def _unblockify(vecs):
    return (vecs.reshape(HT, HT, BLK, BLK).transpose(0, 2, 1, 3)
            .reshape(HT * BLK, HT * BLK))


def _vq_tc_kernel(x_ref, cb_ref, xhat_ref, idx_ref, loss_ref, perp_ref,
                  acc_loss, acc_counts):
    bi = pl.program_id(1)
    img = x_ref[0, 0]
    flat = _blockify(img)
    cb = cb_ref[0]
    rn = jnp.sum(flat * flat, axis=1, keepdims=True)
    cn = jnp.sum(cb * cb, axis=1)[None, :]
    g = jnp.dot(flat, cb.T, preferred_element_type=jnp.float32)
    d = rn - 2.0 * g + cn
    idx = jnp.argmin(d, axis=1)
    onehot = (idx[:, None] == jax.lax.broadcasted_iota(jnp.int32, d.shape, 1)
              ).astype(jnp.float32)
    q = jnp.dot(onehot, cb, preferred_element_type=jnp.float32)
    xhat_ref[0, 0] = _unblockify(q)
    idx_ref[0, 0] = idx[:, None]
    diff = q - flat
    sq = jnp.sum(diff * diff)
    counts = jnp.sum(onehot, axis=0, keepdims=True)

    @pl.when(bi == 0)
    def _init():
        acc_loss[0, 0] = sq
        acc_counts[...] = counts

    @pl.when(bi > 0)
    def _accum():
        acc_loss[0, 0] += sq
        acc_counts[...] += counts

    @pl.when(bi == NB - 1)
    def _final():
        loss_ref[...] = (1.25 / (NB * NV * DIMV) * acc_loss[0, 0]).reshape(1, 1, 1)
        avg = acc_counts[...] * (1.0 / (NB * NV))
        perp_ref[...] = jnp.exp(-jnp.sum(avg * jnp.log(avg + 1e-10))).reshape(1, 1, 1)


def kernel(x, cb0, cb1, cb2, cb3, interpret=False):
    b, c, h, w = x.shape
    cbs = []
    for cb in (cb0, cb1, cb2, cb3):
        k = cb.shape[0]
        if k < KPAD:
            cb = jnp.concatenate(
                [cb, jnp.full((KPAD - k, DIMV), PAD_VAL, cb.dtype)], axis=0)
        cbs.append(cb)
    cb_pad = jnp.stack(cbs, axis=0)

    x_hat, idx, loss, perp = pl.pallas_call(
        _vq_tc_kernel,
        grid=(c, b),
        in_specs=[
            pl.BlockSpec((1, 1, h, w), lambda i, j: (j, i, 0, 0)),
            pl.BlockSpec((1, KPAD, DIMV), lambda i, j: (i, 0, 0)),
        ],
        out_specs=[
            pl.BlockSpec((1, 1, h, w), lambda i, j: (j, i, 0, 0)),
            pl.BlockSpec((1, 1, NV, 1), lambda i, j: (i, j, 0, 0)),
            pl.BlockSpec((1, 1, 1), lambda i, j: (i, 0, 0)),
            pl.BlockSpec((1, 1, 1), lambda i, j: (i, 0, 0)),
        ],
        out_shape=[
            jax.ShapeDtypeStruct((b, c, h, w), jnp.float32),
            jax.ShapeDtypeStruct((c, b, NV, 1), jnp.int32),
            jax.ShapeDtypeStruct((c, 1, 1), jnp.float32),
            jax.ShapeDtypeStruct((c, 1, 1), jnp.float32),
        ],
        scratch_shapes=[
            pltpu.SMEM((1, 1), jnp.float32),
            pltpu.VMEM((1, KPAD), jnp.float32),
        ],
        interpret=interpret,
    )(x, cb_pad)

    indices = idx.reshape(c, b, NV).transpose(1, 0, 2)
    return (x_hat, indices, loss.reshape(c), perp.reshape(c))


# speedup vs baseline: 3.5166x; 1.4699x over previous
import functools

import jax
import jax.numpy as jnp
from jax.experimental import pallas as pl
from jax.experimental.pallas import tpu as pltpu

BLK = 16
HT = 16
NV = 256
DIMV = 256
KPAD = 1024
NB = 8
NC = 4
PAD_VAL = 1e4


def _blockify(img):
    return (img.reshape(HT, BLK, HT, BLK).transpose(0, 2, 1, 3)
            .reshape(NV, DIMV))


def _unblockify(vecs):
    return (vecs.reshape(HT, HT, BLK, BLK).transpose(0, 2, 1, 3)
            .reshape(HT * BLK, HT * BLK))


IMB = 8


def _vq_tc_kernel(x_ref, cb_ref, xhat_ref, idx_ref, loss_ref, perp_ref,
                  acc_loss, acc_counts):
    bi = pl.program_id(1)
    cb = cb_ref[0]
    flat = jnp.concatenate(
        [_blockify(x_ref[m, 0]) for m in range(IMB)], axis=0)
    rn = jnp.sum(flat * flat, axis=1, keepdims=True)
    cn = jnp.sum(cb * cb, axis=1)[None, :]
    g = jnp.dot(flat, cb.T, preferred_element_type=jnp.float32)
    d = rn - 2.0 * g + cn
    idx = jnp.argmin(d, axis=1)
    onehot = (idx[:, None] == jax.lax.broadcasted_iota(jnp.int32, d.shape, 1)
              ).astype(jnp.float32)
    q = jnp.dot(onehot, cb, preferred_element_type=jnp.float32)
    for m in range(IMB):
        xhat_ref[m, 0] = _unblockify(q[m * NV:(m + 1) * NV])
    idx_ref[0, 0] = idx[:, None]
    diff = q - flat
    sq = jnp.sum(diff * diff)
    counts = jnp.sum(onehot, axis=0, keepdims=True)

    @pl.when(bi == 0)
    def _init():
        acc_loss[0, 0] = sq
        acc_counts[...] = counts

    @pl.when(bi > 0)
    def _accum():
        acc_loss[0, 0] += sq
        acc_counts[...] += counts

    @pl.when(bi == NB // IMB - 1)
    def _final():
        loss_ref[...] = (1.25 / (NB * NV * DIMV) * acc_loss[0, 0]).reshape(1, 1, 1)
        avg = acc_counts[...] * (1.0 / (NB * NV))
        perp_ref[...] = jnp.exp(-jnp.sum(avg * jnp.log(avg + 1e-10))).reshape(1, 1, 1)


def kernel(x, cb0, cb1, cb2, cb3, interpret=False):
    b, c, h, w = x.shape
    cbs = []
    for cb in (cb0, cb1, cb2, cb3):
        k = cb.shape[0]
        if k < KPAD:
            cb = jnp.concatenate(
                [cb, jnp.full((KPAD - k, DIMV), PAD_VAL, cb.dtype)], axis=0)
        cbs.append(cb)
    cb_pad = jnp.stack(cbs, axis=0)

    x_hat, idx, loss, perp = pl.pallas_call(
        _vq_tc_kernel,
        grid=(c, b // IMB),
        in_specs=[
            pl.BlockSpec((IMB, 1, h, w), lambda i, j: (j, i, 0, 0)),
            pl.BlockSpec((1, KPAD, DIMV), lambda i, j: (i, 0, 0)),
        ],
        out_specs=[
            pl.BlockSpec((IMB, 1, h, w), lambda i, j: (j, i, 0, 0)),
            pl.BlockSpec((1, 1, IMB * NV, 1), lambda i, j: (i, j, 0, 0)),
            pl.BlockSpec((1, 1, 1), lambda i, j: (i, 0, 0)),
            pl.BlockSpec((1, 1, 1), lambda i, j: (i, 0, 0)),
        ],
        out_shape=[
            jax.ShapeDtypeStruct((b, c, h, w), jnp.float32),
            jax.ShapeDtypeStruct((c, b // IMB, IMB * NV, 1), jnp.int32),
            jax.ShapeDtypeStruct((c, 1, 1), jnp.float32),
            jax.ShapeDtypeStruct((c, 1, 1), jnp.float32),
        ],
        scratch_shapes=[
            pltpu.SMEM((1, 1), jnp.float32),
            pltpu.VMEM((1, KPAD), jnp.float32),
        ],
        interpret=interpret,
    )(x, cb_pad)

    indices = idx.reshape(c, b, NV).transpose(1, 0, 2)
    return (x_hat, indices, loss.reshape(c), perp.reshape(c))


# speedup vs baseline: 3.7417x; 1.0640x over previous
import functools

import jax
import jax.numpy as jnp
from jax.experimental import pallas as pl
from jax.experimental.pallas import tpu as pltpu

BLK = 16
HT = 16
NV = 256
DIMV = 256
KPAD = 1024
NB = 8
NC = 4
PAD_VAL = 1e4


def _blockify(img):
    return (img.reshape(HT, BLK, HT, BLK).transpose(0, 2, 1, 3)
            .reshape(NV, DIMV))


def _unblockify(vecs):
    return (vecs.reshape(HT, HT, BLK, BLK).transpose(0, 2, 1, 3)
            .reshape(HT * BLK, HT * BLK))


IMB = 8


def _vq_tc_kernel(x_ref, cb_ref, xhat_ref, idx_ref, loss_ref, perp_ref,
                  acc_loss, acc_counts):
    bi = pl.program_id(1)
    cb = cb_ref[0]
    flat = jnp.concatenate(
        [_blockify(x_ref[m, 0]) for m in range(IMB)], axis=0)
    rn = jnp.sum(flat * flat, axis=1, keepdims=True)
    cn = jnp.sum(cb * cb, axis=1)[None, :]
    g = jnp.dot(flat, cb.T, preferred_element_type=jnp.float32)
    d = rn - 2.0 * g + cn
    idx = jnp.argmin(d, axis=1)
    onehot = (idx[:, None] == jax.lax.broadcasted_iota(jnp.int32, d.shape, 1)
              ).astype(jnp.float32)
    q = jnp.dot(onehot, cb, preferred_element_type=jnp.float32)
    q_bf = q.astype(jnp.bfloat16)
    for m in range(IMB):
        xhat_ref[m, 0] = _unblockify(q_bf[m * NV:(m + 1) * NV]).astype(jnp.float32)
    idx_ref[0, 0] = idx[:, None]
    diff = q - flat
    sq = jnp.sum(diff * diff)
    counts = jnp.sum(onehot, axis=0, keepdims=True)

    @pl.when(bi == 0)
    def _init():
        acc_loss[0, 0] = sq
        acc_counts[...] = counts

    @pl.when(bi > 0)
    def _accum():
        acc_loss[0, 0] += sq
        acc_counts[...] += counts

    @pl.when(bi == NB // IMB - 1)
    def _final():
        loss_ref[...] = (1.25 / (NB * NV * DIMV) * acc_loss[0, 0]).reshape(1, 1, 1)
        avg = acc_counts[...] * (1.0 / (NB * NV))
        perp_ref[...] = jnp.exp(-jnp.sum(avg * jnp.log(avg + 1e-10))).reshape(1, 1, 1)


def kernel(x, cb0, cb1, cb2, cb3, interpret=False):
    b, c, h, w = x.shape
    cbs = []
    for cb in (cb0, cb1, cb2, cb3):
        k = cb.shape[0]
        if k < KPAD:
            cb = jnp.concatenate(
                [cb, jnp.full((KPAD - k, DIMV), PAD_VAL, cb.dtype)], axis=0)
        cbs.append(cb)
    cb_pad = jnp.stack(cbs, axis=0)

    x_hat, idx, loss, perp = pl.pallas_call(
        _vq_tc_kernel,
        grid=(c, b // IMB),
        in_specs=[
            pl.BlockSpec((IMB, 1, h, w), lambda i, j: (j, i, 0, 0)),
            pl.BlockSpec((1, KPAD, DIMV), lambda i, j: (i, 0, 0)),
        ],
        out_specs=[
            pl.BlockSpec((IMB, 1, h, w), lambda i, j: (j, i, 0, 0)),
            pl.BlockSpec((1, 1, IMB * NV, 1), lambda i, j: (i, j, 0, 0)),
            pl.BlockSpec((1, 1, 1), lambda i, j: (i, 0, 0)),
            pl.BlockSpec((1, 1, 1), lambda i, j: (i, 0, 0)),
        ],
        out_shape=[
            jax.ShapeDtypeStruct((b, c, h, w), jnp.float32),
            jax.ShapeDtypeStruct((c, b // IMB, IMB * NV, 1), jnp.int32),
            jax.ShapeDtypeStruct((c, 1, 1), jnp.float32),
            jax.ShapeDtypeStruct((c, 1, 1), jnp.float32),
        ],
        scratch_shapes=[
            pltpu.SMEM((1, 1), jnp.float32),
            pltpu.VMEM((1, KPAD), jnp.float32),
        ],
        interpret=interpret,
    )(x, cb_pad)

    indices = idx.reshape(c, b, NV).transpose(1, 0, 2)
    return (x_hat, indices, loss.reshape(c), perp.reshape(c))
